# Initial kernel scaffold; baseline (speedup 1.0000x reference)
import jax
import jax.numpy as jnp
from jax.experimental import pallas as pl


def kernel(hm, wh, hps, reg, hm_hp, hp_offset):
    raise NotImplementedError("write your pallas kernel here")


# speedup vs baseline: 1.0084x; 1.0084x over previous
import functools

import jax
import jax.numpy as jnp
from jax.experimental import pallas as pl

B, H, W, K, J = 8, 256, 256, 100, 17
HW = H * W


def _sig_nms_body(x_ref, s_ref, sc_ref):
    x = x_ref[0]
    s = jax.nn.sigmoid(x)
    s_ref[0] = s
    neg = jnp.full((H, 1), -jnp.inf, x.dtype)
    mrow = jnp.maximum(x, jnp.concatenate([x[:, 1:], neg], axis=1))
    mrow = jnp.maximum(mrow, jnp.concatenate([neg, x[:, :-1]], axis=1))
    negr = jnp.full((1, W), -jnp.inf, x.dtype)
    mcol = jnp.maximum(mrow, jnp.concatenate([mrow[1:, :], negr], axis=0))
    mcol = jnp.maximum(mcol, jnp.concatenate([negr, mrow[:-1, :]], axis=0))
    keep = (mcol == x)
    sc_ref[0] = jnp.where(keep, s, 0.0)


def _sig_nms(x):
    b, c = x.shape[0], x.shape[1]
    xf = x.reshape(b * c, H, W)
    out = pl.pallas_call(
        _sig_nms_body,
        grid=(b * c,),
        in_specs=[pl.BlockSpec((1, H, W), lambda i: (i, 0, 0))],
        out_specs=[pl.BlockSpec((1, H, W), lambda i: (i, 0, 0))] * 2,
        out_shape=[jax.ShapeDtypeStruct((b * c, H, W), x.dtype)] * 2,
    )(xf)
    return out[0].reshape(x.shape), out[1].reshape(x.shape)


def _gather_feat(feat, ind):
    b, k = ind.shape
    c = feat.shape[2]
    idx = jnp.broadcast_to(ind[:, :, None], (b, k, c))
    return jnp.take_along_axis(feat, idx, axis=1)


def _transpose_gather(feat, ind):
    b, c, h, w = feat.shape
    feat = jnp.transpose(feat, (0, 2, 3, 1)).reshape(b, h * w, c)
    return _gather_feat(feat, ind)


def kernel(hm, wh, hps, reg, hm_hp, hp_offset):
    hm_s, hm_score_map = _sig_nms(hm)
    hm_hp_s, hp_score_map = _sig_nms(hm_hp)

    b = B
    heat = hm_score_map
    topk_scores, topk_inds = jax.lax.top_k(heat.reshape(b, 1, -1), K)
    topk_inds = topk_inds % (H * W)
    topk_ys = (topk_inds // W).astype(jnp.float32)
    topk_xs = (topk_inds % W).astype(jnp.float32)
    topk_score, topk_ind = jax.lax.top_k(topk_scores.reshape(b, -1), K)
    clses = (topk_ind // K).astype(jnp.float32)
    inds = _gather_feat(topk_inds.reshape(b, -1, 1), topk_ind).reshape(b, K)
    ys = _gather_feat(topk_ys.reshape(b, -1, 1), topk_ind).reshape(b, K)
    xs = _gather_feat(topk_xs.reshape(b, -1, 1), topk_ind).reshape(b, K)
    scores = topk_score
    kps = _transpose_gather(hps, inds)
    kps = kps.at[..., 0::2].add(xs[:, :, None])
    kps = kps.at[..., 1::2].add(ys[:, :, None])
    regg = _transpose_gather(reg, inds)
    xs2 = xs[:, :, None] + regg[:, :, 0:1]
    ys2 = ys[:, :, None] + regg[:, :, 1:2]
    whg = _transpose_gather(wh, inds)
    scores2 = scores[:, :, None]
    clses2 = clses[:, :, None]
    bboxes = jnp.concatenate([
        xs2 - whg[..., 0:1] / 2, ys2 - whg[..., 1:2] / 2,
        xs2 + whg[..., 0:1] / 2, ys2 + whg[..., 1:2] / 2], axis=2)
    thresh = 0.1
    kps = jnp.transpose(kps.reshape(b, K, J, 2), (0, 2, 1, 3))
    reg_kps = jnp.broadcast_to(kps[:, :, :, None, :], (b, J, K, K, 2))
    hm_score, hm_inds = jax.lax.top_k(hp_score_map.reshape(b, J, -1), K)
    hm_inds = hm_inds % (H * W)
    hm_ys = (hm_inds // W).astype(jnp.float32)
    hm_xs = (hm_inds % W).astype(jnp.float32)
    hp_off = _transpose_gather(hp_offset, hm_inds.reshape(b, -1)).reshape(b, J, K, 2)
    hm_xs = hm_xs + hp_off[..., 0]
    hm_ys = hm_ys + hp_off[..., 1]
    mask = (hm_score > thresh).astype(jnp.float32)
    hm_score = (1 - mask) * -1 + mask * hm_score
    hm_ys = (1 - mask) * -10000 + mask * hm_ys
    hm_xs = (1 - mask) * -10000 + mask * hm_xs
    hm_kps = jnp.stack([hm_xs, hm_ys], axis=-1)[:, :, None, :, :]
    hm_kps = jnp.broadcast_to(hm_kps, (b, J, K, K, 2))
    dist = jnp.sqrt(((reg_kps - hm_kps) ** 2).sum(axis=4))
    min_dist = jnp.min(dist, axis=3)
    min_ind = jnp.argmin(dist, axis=3)
    hm_score = jnp.take_along_axis(hm_score, min_ind, axis=2)[..., None]
    min_dist_e = min_dist[..., None]
    gidx = jnp.broadcast_to(min_ind[:, :, :, None, None], (b, J, K, 1, 2))
    hm_kps = jnp.take_along_axis(hm_kps, gidx, axis=3).reshape(b, J, K, 2)
    l = jnp.broadcast_to(bboxes[:, :, 0].reshape(b, 1, K, 1), (b, J, K, 1))
    t = jnp.broadcast_to(bboxes[:, :, 1].reshape(b, 1, K, 1), (b, J, K, 1))
    r = jnp.broadcast_to(bboxes[:, :, 2].reshape(b, 1, K, 1), (b, J, K, 1))
    bo = jnp.broadcast_to(bboxes[:, :, 3].reshape(b, 1, K, 1), (b, J, K, 1))
    mask2 = ((hm_kps[..., 0:1] < l) | (hm_kps[..., 0:1] > r) |
             (hm_kps[..., 1:2] < t) | (hm_kps[..., 1:2] > bo) |
             (hm_score < thresh) | (min_dist_e > jnp.maximum(bo - t, r - l) * 0.3))
    mask2 = jnp.broadcast_to(mask2.astype(jnp.float32), (b, J, K, 2))
    kps = (1 - mask2) * hm_kps + mask2 * kps
    kps = jnp.transpose(kps, (0, 2, 1, 3)).reshape(b, K, J * 2)
    det = jnp.concatenate([bboxes, scores2, kps, clses2], axis=2)
    return (hm_s, wh, hps, reg, hm_hp_s, hp_offset, det)


# speedup vs baseline: 8.0094x; 7.9428x over previous
import functools

import jax
import jax.numpy as jnp
from jax import lax
from jax.experimental import pallas as pl
from jax.experimental.pallas import tpu as pltpu
from jax.experimental.pallas import tpu_sc as plsc

B, H, W, K, J = 8, 256, 256, 100, 17
HW = H * W
NMAPS = B + B * J
NV = HW // 16
CAP = 8192
CAPP = CAP + 16


def _sig_nms_body(x_ref, s_ref, sc_ref):
    x = x_ref[0]
    s = jax.nn.sigmoid(x)
    s_ref[0] = s
    neg = jnp.full((H, 1), -jnp.inf, x.dtype)
    mrow = jnp.maximum(s, jnp.concatenate([s[:, 1:], neg], axis=1))
    mrow = jnp.maximum(mrow, jnp.concatenate([neg, s[:, :-1]], axis=1))
    negr = jnp.full((1, W), -jnp.inf, x.dtype)
    mcol = jnp.maximum(mrow, jnp.concatenate([mrow[1:, :], negr], axis=0))
    mcol = jnp.maximum(mcol, jnp.concatenate([negr, mrow[:-1, :]], axis=0))
    keep = (mcol == s)
    sc_ref[0] = jnp.where(keep, s, 0.0)


def _sig_nms(x):
    b, c = x.shape[0], x.shape[1]
    xf = x.reshape(b * c, H, W)
    out = pl.pallas_call(
        _sig_nms_body,
        grid=(b * c,),
        in_specs=[pl.BlockSpec((1, H, W), lambda i: (i, 0, 0))],
        out_specs=[pl.BlockSpec((1, H, W), lambda i: (i, 0, 0))] * 2,
        out_shape=[jax.ShapeDtypeStruct((b * c, H, W), x.dtype)] * 2,
    )(xf)
    return out[0].reshape(x.shape), out[1].reshape(b * c, HW)


def _rung(k):
    t = jnp.where(k == 0, jnp.float32(0.999), jnp.float32(1e-8))
    t = jnp.where(k == 1, jnp.float32(0.99), t)
    t = jnp.where(k == 2, jnp.float32(0.9), t)
    return t


def _sc_topk(maps):
    mesh = plsc.VectorSubcoreMesh(core_axis_name="c", subcore_axis_name="s")

    @functools.partial(
        pl.kernel,
        out_type=[
            jax.ShapeDtypeStruct((NMAPS, 128), jnp.float32),
            jax.ShapeDtypeStruct((NMAPS, 128), jnp.int32),
        ],
        mesh=mesh,
        compiler_params=pltpu.CompilerParams(needs_layout_passes=False),
        scratch_types=[
            pltpu.VMEM((HW,), jnp.float32),
            pltpu.VMEM((CAPP,), jnp.float32),
            pltpu.VMEM((CAPP,), jnp.int32),
            pltpu.VMEM((128,), jnp.float32),
            pltpu.VMEM((128,), jnp.int32),
            pltpu.VMEM((128,), jnp.float32),
            pltpu.VMEM((128,), jnp.int32),
            pltpu.VMEM((112,), jnp.int32),
            pltpu.VMEM((128,), jnp.float32),
            pltpu.VMEM((128,), jnp.int32),
        ],
    )
    def topk_kernel(maps_hbm, outv_hbm, outi_hbm,
                    map_v, cand_v, cidx_v, tie_v, tidx_v,
                    fin_v, fidx_v, rank_v, outv_v, outi_v):
        w = lax.axis_index("s") * 2 + lax.axis_index("c")
        laneiota = lax.iota(jnp.int32, 16)

        def compact_pass(t, nvec):
            def body(i, cnt):
                v = map_v[pl.ds(i * 16, 16)]
                msk = v >= t
                n = jnp.sum(msk.astype(jnp.int32))
                off = jnp.minimum(cnt, CAP)
                plsc.store_compressed(cand_v.at[pl.ds(off, 16)], v, mask=msk)
                idxv = laneiota + i * 16
                plsc.store_compressed(cidx_v.at[pl.ds(off, 16)], idxv,
                                      mask=msk)
                return cnt + n
            return lax.fori_loop(0, nvec, body, jnp.int32(0))

        def count_ge(tb, cnt, nv_cand, nv_map):
            def cbody(i, acc):
                v = cand_v[pl.ds(i * 16, 16)]
                bits = plsc.bitcast(v, jnp.int32)
                valid = (i * 16 + laneiota) < cnt
                m = (bits >= tb) & valid
                return acc + jnp.sum(m.astype(jnp.int32))
            acc = lax.fori_loop(0, nv_cand, cbody, jnp.int32(0))

            def mbody(i, acc):
                v = map_v[pl.ds(i * 16, 16)]
                bits = plsc.bitcast(v, jnp.int32)
                m = bits >= tb
                return acc + jnp.sum(m.astype(jnp.int32))
            return lax.fori_loop(0, nv_map, mbody, acc)

        def process(g):
            pltpu.sync_copy(maps_hbm.at[g], map_v)

            cnt = compact_pass(_rung(jnp.int32(2)), NV)

            def step(_, st):
                k, c = st
                ok = (c >= K) & (c <= CAP)
                k2 = jnp.where(c > CAP, k - 1, jnp.where(c < K, k + 1, k))
                live = (~ok) & (k2 >= 0) & (k2 <= 3)
                c2 = compact_pass(_rung(k2), jnp.where(live, NV, 0))
                return (jnp.where(ok, k, k2), jnp.where(live, c2, c))
            _, cnt = lax.fori_loop(0, 3, step, (jnp.int32(2), cnt))
            fb = ~((cnt >= K) & (cnt <= CAP))

            nv_cand = jnp.where(fb, 0, (cnt + 15) // 16)
            nv_map = jnp.where(fb, NV, 0)

            def bbody(_, st):
                lo, hi = st
                mid = lo + (hi - lo) // 2
                n = count_ge(mid, cnt, nv_cand, nv_map)
                return (jnp.where(n >= K, mid, lo),
                        jnp.where(n >= K, hi, mid))
            vkb, _ = lax.fori_loop(
                0, 31, bbody,
                (jnp.where(fb, jnp.int32(0), jnp.int32(1)),
                 jnp.int32(0x3F800001)))

            def hc_body(i, p):
                v = cand_v[pl.ds(i * 16, 16)]
                ix = cidx_v[pl.ds(i * 16, 16)]
                bits = plsc.bitcast(v, jnp.int32)
                valid = (i * 16 + laneiota) < cnt
                m = (bits > vkb) & valid
                plsc.store_compressed(fin_v.at[pl.ds(p, 16)], v, mask=m)
                plsc.store_compressed(fidx_v.at[pl.ds(p, 16)], ix, mask=m)
                return p + jnp.sum(m.astype(jnp.int32))
            nhi = lax.fori_loop(0, nv_cand, hc_body, jnp.int32(0))

            def hm_body(i, p):
                v = map_v[pl.ds(i * 16, 16)]
                bits = plsc.bitcast(v, jnp.int32)
                m = bits > vkb
                plsc.store_compressed(fin_v.at[pl.ds(p, 16)], v, mask=m)
                plsc.store_compressed(fidx_v.at[pl.ds(p, 16)],
                                      laneiota + i * 16, mask=m)
                return p + jnp.sum(m.astype(jnp.int32))
            nhi = lax.fori_loop(0, nv_map, hm_body, nhi)

            def tc_body(i, p):
                v = cand_v[pl.ds(i * 16, 16)]
                ix = cidx_v[pl.ds(i * 16, 16)]
                bits = plsc.bitcast(v, jnp.int32)
                valid = (i * 16 + laneiota) < cnt
                m = (bits == vkb) & valid
                off = jnp.minimum(p, 100)
                plsc.store_compressed(tie_v.at[pl.ds(off, 16)], v, mask=m)
                plsc.store_compressed(tidx_v.at[pl.ds(off, 16)], ix, mask=m)
                return p + jnp.sum(m.astype(jnp.int32))
            nt = lax.fori_loop(0, nv_cand, tc_body, jnp.int32(0))

            def tm_body(i, p):
                v = map_v[pl.ds(i * 16, 16)]
                bits = plsc.bitcast(v, jnp.int32)
                m = bits == vkb
                off = jnp.minimum(p, 100)
                plsc.store_compressed(tie_v.at[pl.ds(off, 16)], v, mask=m)
                plsc.store_compressed(tidx_v.at[pl.ds(off, 16)],
                                      laneiota + i * 16, mask=m)
                return p + jnp.sum(m.astype(jnp.int32))
            lax.fori_loop(0, nv_map, tm_body, nt)

            for t in range(7):
                p = t * 16 + laneiota
                cur_v = fin_v[pl.ds(t * 16, 16)]
                cur_i = fidx_v[pl.ds(t * 16, 16)]
                src = jnp.maximum(p - nhi, 0)
                tv = plsc.load_gather(tie_v, [src])
                ti = plsc.load_gather(tidx_v, [src])
                in_hi = p < nhi
                in_tie = p < K
                nv = jnp.where(in_hi, cur_v,
                               jnp.where(in_tie, tv, jnp.float32(-1.0)))
                ni = jnp.where(in_hi, cur_i,
                               jnp.where(in_tie, ti, jnp.int32(0x7FFFFFFF)))
                fin_v[pl.ds(t * 16, 16)] = nv
                fidx_v[pl.ds(t * 16, 16)] = ni
                rank_v[pl.ds(t * 16, 16)] = jnp.zeros((16,), jnp.int32)

            def rbody(j, _):
                vj = fin_v[pl.ds(j, 16)][0]
                ij = fidx_v[pl.ds(j, 16)][0]
                for t in range(7):
                    fv = fin_v[pl.ds(t * 16, 16)]
                    fi = fidx_v[pl.ds(t * 16, 16)]
                    before = (vj > fv) | ((vj == fv) & (ij < fi))
                    acc = rank_v[pl.ds(t * 16, 16)]
                    rank_v[pl.ds(t * 16, 16)] = acc + before.astype(jnp.int32)
                return 0
            lax.fori_loop(0, K, rbody, 0)

            for t in range(7):
                r = rank_v[pl.ds(t * 16, 16)]
                m = r < K
                rc = jnp.minimum(r, jnp.int32(127))
                plsc.store_scatter(outv_v, [rc], fin_v[pl.ds(t * 16, 16)],
                                   mask=m)
                plsc.store_scatter(outi_v, [rc], fidx_v[pl.ds(t * 16, 16)],
                                   mask=m)

            pltpu.sync_copy(outv_v, outv_hbm.at[g])
            pltpu.sync_copy(outi_v, outi_hbm.at[g])

        def round_body(r, _):
            process(w + 32 * r)
            return 0
        nrounds = jnp.where(w < NMAPS - 128, 5, 4)
        lax.fori_loop(0, nrounds, round_body, 0)

    return topk_kernel(maps)


def _gather_feat(feat, ind):
    b, k = ind.shape
    c = feat.shape[2]
    idx = jnp.broadcast_to(ind[:, :, None], (b, k, c))
    return jnp.take_along_axis(feat, idx, axis=1)


def _transpose_gather(feat, ind):
    b, c, h, w = feat.shape
    feat = jnp.transpose(feat, (0, 2, 3, 1)).reshape(b, h * w, c)
    return _gather_feat(feat, ind)


def kernel(hm, wh, hps, reg, hm_hp, hp_offset):
    hm_s, hm_scores = _sig_nms(hm)
    hm_hp_s, hp_scores = _sig_nms(hm_hp)

    maps = jnp.concatenate([hm_scores, hp_scores], axis=0)
    vals, idxs = _sc_topk(maps)
    b = B
    scores = vals[:B, :K]
    inds = idxs[:B, :K]
    hm_score = vals[B:, :K].reshape(b, J, K)
    hm_inds = idxs[B:, :K].reshape(b, J, K)

    ys = (inds // W).astype(jnp.float32)
    xs = (inds % W).astype(jnp.float32)
    clses2 = jnp.zeros((b, K, 1), jnp.float32)

    kps = _transpose_gather(hps, inds)
    kps = kps.at[..., 0::2].add(xs[:, :, None])
    kps = kps.at[..., 1::2].add(ys[:, :, None])
    regg = _transpose_gather(reg, inds)
    xs2 = xs[:, :, None] + regg[:, :, 0:1]
    ys2 = ys[:, :, None] + regg[:, :, 1:2]
    whg = _transpose_gather(wh, inds)
    scores2 = scores[:, :, None]
    bboxes = jnp.concatenate([
        xs2 - whg[..., 0:1] / 2, ys2 - whg[..., 1:2] / 2,
        xs2 + whg[..., 0:1] / 2, ys2 + whg[..., 1:2] / 2], axis=2)
    thresh = 0.1
    kps = jnp.transpose(kps.reshape(b, K, J, 2), (0, 2, 1, 3))
    reg_kps = jnp.broadcast_to(kps[:, :, :, None, :], (b, J, K, K, 2))
    hm_ys = (hm_inds // W).astype(jnp.float32)
    hm_xs = (hm_inds % W).astype(jnp.float32)
    hp_off = _transpose_gather(hp_offset, hm_inds.reshape(b, -1)).reshape(b, J, K, 2)
    hm_xs = hm_xs + hp_off[..., 0]
    hm_ys = hm_ys + hp_off[..., 1]
    mask = (hm_score > thresh).astype(jnp.float32)
    hm_score = (1 - mask) * -1 + mask * hm_score
    hm_ys = (1 - mask) * -10000 + mask * hm_ys
    hm_xs = (1 - mask) * -10000 + mask * hm_xs
    hm_kps = jnp.stack([hm_xs, hm_ys], axis=-1)[:, :, None, :, :]
    hm_kps = jnp.broadcast_to(hm_kps, (b, J, K, K, 2))
    dist = jnp.sqrt(((reg_kps - hm_kps) ** 2).sum(axis=4))
    min_dist = jnp.min(dist, axis=3)
    min_ind = jnp.argmin(dist, axis=3)
    hm_score = jnp.take_along_axis(hm_score, min_ind, axis=2)[..., None]
    min_dist_e = min_dist[..., None]
    gidx = jnp.broadcast_to(min_ind[:, :, :, None, None], (b, J, K, 1, 2))
    hm_kps = jnp.take_along_axis(hm_kps, gidx, axis=3).reshape(b, J, K, 2)
    l = jnp.broadcast_to(bboxes[:, :, 0].reshape(b, 1, K, 1), (b, J, K, 1))
    t = jnp.broadcast_to(bboxes[:, :, 1].reshape(b, 1, K, 1), (b, J, K, 1))
    r = jnp.broadcast_to(bboxes[:, :, 2].reshape(b, 1, K, 1), (b, J, K, 1))
    bo = jnp.broadcast_to(bboxes[:, :, 3].reshape(b, 1, K, 1), (b, J, K, 1))
    mask2 = ((hm_kps[..., 0:1] < l) | (hm_kps[..., 0:1] > r) |
             (hm_kps[..., 1:2] < t) | (hm_kps[..., 1:2] > bo) |
             (hm_score < thresh) | (min_dist_e > jnp.maximum(bo - t, r - l) * 0.3))
    mask2 = jnp.broadcast_to(mask2.astype(jnp.float32), (b, J, K, 2))
    kps = (1 - mask2) * hm_kps + mask2 * kps
    kps = jnp.transpose(kps, (0, 2, 1, 3)).reshape(b, K, J * 2)
    det = jnp.concatenate([bboxes, scores2, kps, clses2], axis=2)
    return (hm_s, wh, hps, reg, hm_hp_s, hp_offset, det)


# speedup vs baseline: 11.2000x; 1.3984x over previous
import functools

import jax
import jax.numpy as jnp
from jax import lax
from jax.experimental import pallas as pl
from jax.experimental.pallas import tpu as pltpu
from jax.experimental.pallas import tpu_sc as plsc

B, H, W, K, J = 8, 256, 256, 100, 17
HW = H * W
NMAPS = B + B * J
NV = HW // 16
CAP = 8192
CAPP = CAP + 16


def _sig_nms_body(x_ref, s_ref, sc_ref):
    x = x_ref[0]
    s = jax.nn.sigmoid(x)
    s_ref[0] = s
    neg = jnp.full((H, 1), -jnp.inf, x.dtype)
    mrow = jnp.maximum(s, jnp.concatenate([s[:, 1:], neg], axis=1))
    mrow = jnp.maximum(mrow, jnp.concatenate([neg, s[:, :-1]], axis=1))
    negr = jnp.full((1, W), -jnp.inf, x.dtype)
    mcol = jnp.maximum(mrow, jnp.concatenate([mrow[1:, :], negr], axis=0))
    mcol = jnp.maximum(mcol, jnp.concatenate([negr, mrow[:-1, :]], axis=0))
    keep = (mcol == s)
    sc_ref[0] = jnp.where(keep, s, 0.0)


def _sig_nms(x):
    b, c = x.shape[0], x.shape[1]
    xf = x.reshape(b * c, H, W)
    out = pl.pallas_call(
        _sig_nms_body,
        grid=(b * c,),
        in_specs=[pl.BlockSpec((1, H, W), lambda i: (i, 0, 0))],
        out_specs=[pl.BlockSpec((1, H, W), lambda i: (i, 0, 0))] * 2,
        out_shape=[jax.ShapeDtypeStruct((b * c, H, W), x.dtype)] * 2,
    )(xf)
    return out[0].reshape(x.shape), out[1].reshape(b * c, HW)


def _rung(k):
    t = jnp.where(k == 0, jnp.float32(0.999), jnp.float32(1e-8))
    t = jnp.where(k == 1, jnp.float32(0.99), t)
    t = jnp.where(k == 2, jnp.float32(0.9), t)
    return t


def _sc_topk(hm_sc, hp_sc):
    mesh = plsc.VectorSubcoreMesh(core_axis_name="c", subcore_axis_name="s")

    @functools.partial(
        pl.kernel,
        out_type=[
            jax.ShapeDtypeStruct((NMAPS, 128), jnp.float32),
            jax.ShapeDtypeStruct((NMAPS, 128), jnp.int32),
        ],
        mesh=mesh,
        compiler_params=pltpu.CompilerParams(needs_layout_passes=False),
        scratch_types=[
            pltpu.VMEM((HW,), jnp.float32),
            pltpu.VMEM((CAPP,), jnp.float32),
            pltpu.VMEM((CAPP,), jnp.int32),
            pltpu.VMEM((128,), jnp.float32),
            pltpu.VMEM((128,), jnp.int32),
            pltpu.VMEM((128,), jnp.float32),
            pltpu.VMEM((128,), jnp.int32),
            pltpu.VMEM((112,), jnp.int32),
            pltpu.VMEM((128,), jnp.float32),
            pltpu.VMEM((128,), jnp.int32),
        ],
    )
    def topk_kernel(hm_hbm, hp_hbm, outv_hbm, outi_hbm,
                    map_v, cand_v, cidx_v, tie_v, tidx_v,
                    fin_v, fidx_v, rank_v, outv_v, outi_v):
        w = lax.axis_index("s") * 2 + lax.axis_index("c")
        laneiota = lax.iota(jnp.int32, 16)

        def compact_pass(t, nvec):
            def body(i, cnt):
                v = map_v[pl.ds(i * 16, 16)]
                msk = v >= t
                n = plsc.all_reduce_population_count(msk)[0]

                @pl.when(n > 0)
                def _():
                    off = jnp.minimum(cnt, CAP)
                    plsc.store_compressed(cand_v.at[pl.ds(off, 16)], v,
                                          mask=msk)
                    idxv = laneiota + i * 16
                    plsc.store_compressed(cidx_v.at[pl.ds(off, 16)], idxv,
                                          mask=msk)
                return cnt + n
            return lax.fori_loop(0, nvec, body, jnp.int32(0))

        def count_ge(tb, cnt, nv_cand, nv_map):
            def cbody(i, acc):
                v = cand_v[pl.ds(i * 16, 16)]
                bits = plsc.bitcast(v, jnp.int32)
                valid = (i * 16 + laneiota) < cnt
                m = (bits >= tb) & valid
                return acc + plsc.all_reduce_population_count(m)[0]
            acc = lax.fori_loop(0, nv_cand, cbody, jnp.int32(0))

            def mbody(i, acc):
                v = map_v[pl.ds(i * 16, 16)]
                bits = plsc.bitcast(v, jnp.int32)
                m = bits >= tb
                return acc + plsc.all_reduce_population_count(m)[0]
            return lax.fori_loop(0, nv_map, mbody, acc)

        def process(g):
            @pl.when(g < B)
            def _():
                pltpu.sync_copy(hm_hbm.at[jnp.minimum(g, B - 1)], map_v)

            @pl.when(g >= B)
            def _():
                pltpu.sync_copy(hp_hbm.at[jnp.maximum(g - B, 0)], map_v)

            cnt = compact_pass(_rung(jnp.int32(2)), NV)

            def step(_, st):
                k, c = st
                ok = (c >= K) & (c <= CAP)
                k2 = jnp.where(c > CAP, k - 1, jnp.where(c < K, k + 1, k))
                live = (~ok) & (k2 >= 0) & (k2 <= 3)
                c2 = compact_pass(_rung(k2), jnp.where(live, NV, 0))
                return (jnp.where(ok, k, k2), jnp.where(live, c2, c))
            _, cnt = lax.fori_loop(0, 3, step, (jnp.int32(2), cnt))
            fb = ~((cnt >= K) & (cnt <= CAP))

            nv_cand = jnp.where(fb, 0, (cnt + 15) // 16)
            nv_map = jnp.where(fb, NV, 0)

            def bbody(_, st):
                lo, hi = st
                mid = lo + (hi - lo) // 2
                n = count_ge(mid, cnt, nv_cand, nv_map)
                return (jnp.where(n >= K, mid, lo),
                        jnp.where(n >= K, hi, mid))
            vkb, _ = lax.fori_loop(
                0, 31, bbody,
                (jnp.where(fb, jnp.int32(0), jnp.int32(1)),
                 jnp.int32(0x3F800001)))

            def hc_body(i, p):
                v = cand_v[pl.ds(i * 16, 16)]
                ix = cidx_v[pl.ds(i * 16, 16)]
                bits = plsc.bitcast(v, jnp.int32)
                valid = (i * 16 + laneiota) < cnt
                m = (bits > vkb) & valid
                plsc.store_compressed(fin_v.at[pl.ds(p, 16)], v, mask=m)
                plsc.store_compressed(fidx_v.at[pl.ds(p, 16)], ix, mask=m)
                return p + plsc.all_reduce_population_count(m)[0]
            nhi = lax.fori_loop(0, nv_cand, hc_body, jnp.int32(0))

            def hm_body(i, p):
                v = map_v[pl.ds(i * 16, 16)]
                bits = plsc.bitcast(v, jnp.int32)
                m = bits > vkb
                plsc.store_compressed(fin_v.at[pl.ds(p, 16)], v, mask=m)
                plsc.store_compressed(fidx_v.at[pl.ds(p, 16)],
                                      laneiota + i * 16, mask=m)
                return p + plsc.all_reduce_population_count(m)[0]
            nhi = lax.fori_loop(0, nv_map, hm_body, nhi)

            def tc_body(i, p):
                v = cand_v[pl.ds(i * 16, 16)]
                ix = cidx_v[pl.ds(i * 16, 16)]
                bits = plsc.bitcast(v, jnp.int32)
                valid = (i * 16 + laneiota) < cnt
                m = (bits == vkb) & valid
                off = jnp.minimum(p, 100)
                plsc.store_compressed(tie_v.at[pl.ds(off, 16)], v, mask=m)
                plsc.store_compressed(tidx_v.at[pl.ds(off, 16)], ix, mask=m)
                return p + plsc.all_reduce_population_count(m)[0]
            nt = lax.fori_loop(0, nv_cand, tc_body, jnp.int32(0))

            def tm_body(i, p):
                v = map_v[pl.ds(i * 16, 16)]
                bits = plsc.bitcast(v, jnp.int32)
                m = bits == vkb
                off = jnp.minimum(p, 100)
                plsc.store_compressed(tie_v.at[pl.ds(off, 16)], v, mask=m)
                plsc.store_compressed(tidx_v.at[pl.ds(off, 16)],
                                      laneiota + i * 16, mask=m)
                return p + plsc.all_reduce_population_count(m)[0]
            lax.fori_loop(0, nv_map, tm_body, nt)

            for t in range(7):
                p = t * 16 + laneiota
                cur_v = fin_v[pl.ds(t * 16, 16)]
                cur_i = fidx_v[pl.ds(t * 16, 16)]
                src = jnp.maximum(p - nhi, 0)
                tv = plsc.load_gather(tie_v, [src])
                ti = plsc.load_gather(tidx_v, [src])
                in_hi = p < nhi
                in_tie = p < K
                nv = jnp.where(in_hi, cur_v,
                               jnp.where(in_tie, tv, jnp.float32(-1.0)))
                ni = jnp.where(in_hi, cur_i,
                               jnp.where(in_tie, ti, jnp.int32(0x7FFFFFFF)))
                fin_v[pl.ds(t * 16, 16)] = nv
                fidx_v[pl.ds(t * 16, 16)] = ni
                rank_v[pl.ds(t * 16, 16)] = jnp.zeros((16,), jnp.int32)

            def rbody(j, _):
                vj = fin_v[pl.ds(j, 16)][0]
                ij = fidx_v[pl.ds(j, 16)][0]
                for t in range(7):
                    fv = fin_v[pl.ds(t * 16, 16)]
                    fi = fidx_v[pl.ds(t * 16, 16)]
                    before = (vj > fv) | ((vj == fv) & (ij < fi))
                    acc = rank_v[pl.ds(t * 16, 16)]
                    rank_v[pl.ds(t * 16, 16)] = acc + before.astype(jnp.int32)
                return 0
            lax.fori_loop(0, K, rbody, 0)

            for t in range(7):
                r = rank_v[pl.ds(t * 16, 16)]
                m = r < K
                rc = jnp.minimum(r, jnp.int32(127))
                plsc.store_scatter(outv_v, [rc], fin_v[pl.ds(t * 16, 16)],
                                   mask=m)
                plsc.store_scatter(outi_v, [rc], fidx_v[pl.ds(t * 16, 16)],
                                   mask=m)

            pltpu.sync_copy(outv_v, outv_hbm.at[g])
            pltpu.sync_copy(outi_v, outi_hbm.at[g])

        def round_body(r, _):
            process(w + 32 * r)
            return 0
        nrounds = jnp.where(w < NMAPS - 128, 5, 4)
        lax.fori_loop(0, nrounds, round_body, 0)

    return topk_kernel(hm_sc, hp_sc)


def _gather_feat(feat, ind):
    b, k = ind.shape
    c = feat.shape[2]
    idx = jnp.broadcast_to(ind[:, :, None], (b, k, c))
    return jnp.take_along_axis(feat, idx, axis=1)


def _transpose_gather(feat, ind):
    b, c, h, w = feat.shape
    feat = jnp.transpose(feat, (0, 2, 3, 1)).reshape(b, h * w, c)
    return _gather_feat(feat, ind)


def kernel(hm, wh, hps, reg, hm_hp, hp_offset):
    hm_s, hm_scores = _sig_nms(hm)
    hm_hp_s, hp_scores = _sig_nms(hm_hp)

    vals, idxs = _sc_topk(hm_scores, hp_scores)
    b = B
    scores = vals[:B, :K]
    inds = idxs[:B, :K]
    hm_score = vals[B:, :K].reshape(b, J, K)
    hm_inds = idxs[B:, :K].reshape(b, J, K)

    ys = (inds // W).astype(jnp.float32)
    xs = (inds % W).astype(jnp.float32)
    clses2 = jnp.zeros((b, K, 1), jnp.float32)

    kps = _transpose_gather(hps, inds)
    kps = kps.at[..., 0::2].add(xs[:, :, None])
    kps = kps.at[..., 1::2].add(ys[:, :, None])
    regg = _transpose_gather(reg, inds)
    xs2 = xs[:, :, None] + regg[:, :, 0:1]
    ys2 = ys[:, :, None] + regg[:, :, 1:2]
    whg = _transpose_gather(wh, inds)
    scores2 = scores[:, :, None]
    bboxes = jnp.concatenate([
        xs2 - whg[..., 0:1] / 2, ys2 - whg[..., 1:2] / 2,
        xs2 + whg[..., 0:1] / 2, ys2 + whg[..., 1:2] / 2], axis=2)
    thresh = 0.1
    kps = jnp.transpose(kps.reshape(b, K, J, 2), (0, 2, 1, 3))
    reg_kps = jnp.broadcast_to(kps[:, :, :, None, :], (b, J, K, K, 2))
    hm_ys = (hm_inds // W).astype(jnp.float32)
    hm_xs = (hm_inds % W).astype(jnp.float32)
    hp_off = _transpose_gather(hp_offset, hm_inds.reshape(b, -1)).reshape(b, J, K, 2)
    hm_xs = hm_xs + hp_off[..., 0]
    hm_ys = hm_ys + hp_off[..., 1]
    mask = (hm_score > thresh).astype(jnp.float32)
    hm_score = (1 - mask) * -1 + mask * hm_score
    hm_ys = (1 - mask) * -10000 + mask * hm_ys
    hm_xs = (1 - mask) * -10000 + mask * hm_xs
    hm_kps = jnp.stack([hm_xs, hm_ys], axis=-1)[:, :, None, :, :]
    hm_kps = jnp.broadcast_to(hm_kps, (b, J, K, K, 2))
    dist = jnp.sqrt(((reg_kps - hm_kps) ** 2).sum(axis=4))
    min_dist = jnp.min(dist, axis=3)
    min_ind = jnp.argmin(dist, axis=3)
    hm_score = jnp.take_along_axis(hm_score, min_ind, axis=2)[..., None]
    min_dist_e = min_dist[..., None]
    gidx = jnp.broadcast_to(min_ind[:, :, :, None, None], (b, J, K, 1, 2))
    hm_kps = jnp.take_along_axis(hm_kps, gidx, axis=3).reshape(b, J, K, 2)
    l = jnp.broadcast_to(bboxes[:, :, 0].reshape(b, 1, K, 1), (b, J, K, 1))
    t = jnp.broadcast_to(bboxes[:, :, 1].reshape(b, 1, K, 1), (b, J, K, 1))
    r = jnp.broadcast_to(bboxes[:, :, 2].reshape(b, 1, K, 1), (b, J, K, 1))
    bo = jnp.broadcast_to(bboxes[:, :, 3].reshape(b, 1, K, 1), (b, J, K, 1))
    mask2 = ((hm_kps[..., 0:1] < l) | (hm_kps[..., 0:1] > r) |
             (hm_kps[..., 1:2] < t) | (hm_kps[..., 1:2] > bo) |
             (hm_score < thresh) | (min_dist_e > jnp.maximum(bo - t, r - l) * 0.3))
    mask2 = jnp.broadcast_to(mask2.astype(jnp.float32), (b, J, K, 2))
    kps = (1 - mask2) * hm_kps + mask2 * kps
    kps = jnp.transpose(kps, (0, 2, 1, 3)).reshape(b, K, J * 2)
    det = jnp.concatenate([bboxes, scores2, kps, clses2], axis=2)
    return (hm_s, wh, hps, reg, hm_hp_s, hp_offset, det)
